# baseline (speedup 1.0000x reference)
import functools

import jax
import jax.numpy as jnp
from jax import lax
from jax.experimental import pallas as pl
from jax.experimental.pallas import tpu as pltpu

NTILES = 32
F32 = jnp.float32


def _wg_body(w_ref, wih_ref, o_ref):
    o_ref[...] = lax.dot_general(
        w_ref[0], wih_ref[...], (((1,), (1,)), ((), ())),
        preferred_element_type=F32)[None]


def _fold_weights(W, wih):
    L, Din, _ = W.shape
    G = wih.shape[0]
    return pl.pallas_call(
        _wg_body,
        grid=(L,),
        in_specs=[
            pl.BlockSpec((1, Din, Din), lambda i: (i, 0, 0)),
            pl.BlockSpec((G, Din), lambda i: (0, 0)),
        ],
        out_specs=pl.BlockSpec((1, Din, G), lambda i: (i, 0, 0)),
        out_shape=jax.ShapeDtypeStruct((L, Din, G), F32),
    )(W, wih)


def _gru_body(q_ref, h_ref, wg_ref, whh_ref, bih_ref, bhh_ref, o_ref, *, H):
    q = q_ref[...]
    h = h_ref[...]
    gi = jnp.dot(q, wg_ref[...], preferred_element_type=F32) + bih_ref[...]
    gh = lax.dot_general(h, whh_ref[...], (((1,), (1,)), ((), ())),
                         preferred_element_type=F32) + bhh_ref[...]
    i_r, i_z, i_n = gi[:, :H], gi[:, H:2 * H], gi[:, 2 * H:]
    h_r, h_z, h_n = gh[:, :H], gh[:, H:2 * H], gh[:, 2 * H:]
    r = jax.nn.sigmoid(i_r + h_r)
    z = jax.nn.sigmoid(i_z + h_z)
    n = jnp.tanh(i_n + r * h_n)
    o_ref[...] = (1.0 - z) * n + z * h


def _gru_step(q, h, wg, whh, bih, bhh, block_rows):
    Np, H = h.shape
    G = 3 * H
    return pl.pallas_call(
        functools.partial(_gru_body, H=H),
        grid=(Np // block_rows,),
        in_specs=[
            pl.BlockSpec((block_rows, H), lambda i: (i, 0)),
            pl.BlockSpec((block_rows, H), lambda i: (i, 0)),
            pl.BlockSpec((H, G), lambda i: (0, 0)),
            pl.BlockSpec((G, H), lambda i: (0, 0)),
            pl.BlockSpec((1, G), lambda i: (0, 0)),
            pl.BlockSpec((1, G), lambda i: (0, 0)),
        ],
        out_specs=pl.BlockSpec((block_rows, H), lambda i: (i, 0)),
        out_shape=jax.ShapeDtypeStruct((Np, H), F32),
    )(q, h, wg, whh, bih.reshape(1, G), bhh.reshape(1, G))


def _lstm_body(x_ref, wih_ref, b_ref, o_ref, *, H):
    g = lax.dot_general(x_ref[...], wih_ref[...], (((1,), (1,)), ((), ())),
                        preferred_element_type=F32) + b_ref[...]
    i, gg, o = g[:, :H], g[:, 2 * H:3 * H], g[:, 3 * H:]
    c = jax.nn.sigmoid(i) * jnp.tanh(gg)
    o_ref[...] = jax.nn.sigmoid(o) * jnp.tanh(c)


def _lstm_step(x, wih, bih, bhh, block_rows):
    Np, Din = x.shape
    H = wih.shape[0] // 4
    return pl.pallas_call(
        functools.partial(_lstm_body, H=H),
        grid=(Np // block_rows,),
        in_specs=[
            pl.BlockSpec((block_rows, Din), lambda i: (i, 0)),
            pl.BlockSpec((4 * H, Din), lambda i: (0, 0)),
            pl.BlockSpec((1, 4 * H), lambda i: (0, 0)),
        ],
        out_specs=pl.BlockSpec((block_rows, H), lambda i: (i, 0)),
        out_shape=jax.ShapeDtypeStruct((Np, H), F32),
    )(x, wih, (bih + bhh).reshape(1, 4 * H))


def _final_body(x_ref, lw_ref, lb_ref, o_ref):
    y = jnp.maximum(x_ref[...], 0.0)
    o_ref[...] = lax.dot_general(y, lw_ref[...], (((1,), (1,)), ((), ())),
                                 preferred_element_type=F32) + lb_ref[...]


def _final_linear(x, lin_w, lin_b, block_rows):
    Np, H = x.shape
    lw = jnp.broadcast_to(lin_w, (128, H))
    lb = jnp.broadcast_to(lin_b.reshape(1, 1), (1, 128))
    return pl.pallas_call(
        _final_body,
        grid=(Np // block_rows,),
        in_specs=[
            pl.BlockSpec((block_rows, H), lambda i: (i, 0)),
            pl.BlockSpec((128, H), lambda i: (0, 0)),
            pl.BlockSpec((1, 128), lambda i: (0, 0)),
        ],
        out_specs=pl.BlockSpec((block_rows, 128), lambda i: (i, 0)),
        out_shape=jax.ShapeDtypeStruct((Np, 128), F32),
    )(x, lw, lb)


def _spmm(h, src_s, dst_s, ew2, npad):
    msg = jnp.take(h, src_s, axis=0) * ew2[:, None]
    return jax.ops.segment_sum(msg, dst_s, num_segments=npad)


def kernel(x, edge_index, edge_weight, W1, gru1_wih, gru1_whh, gru1_bih,
           gru1_bhh, lstm1_wih, lstm1_whh, lstm1_bih, lstm1_bhh, W2,
           gru2_wih, gru2_whh, gru2_bih, gru2_bhh, lstm2_wih, lstm2_whh,
           lstm2_bih, lstm2_bhh, lin_w, lin_b):
    N, D = x.shape
    E = edge_weight.shape[0]
    L = W1.shape[0]
    npad = ((N + NTILES * 8 - 1) // (NTILES * 8)) * (NTILES * 8)
    br1 = 1024 if npad % 1024 == 0 else npad
    br2 = 2048 if npad % 2048 == 0 else npad

    src = edge_index[0]
    dst = edge_index[1]
    order = jnp.argsort(dst)
    src_s = src[order]
    dst_s = dst[order]
    ew_s = edge_weight[order]

    deg = jnp.clip(jax.ops.segment_sum(jnp.ones_like(ew_s), dst_s,
                                       num_segments=npad), 1.0, None)
    ew2 = ew_s / deg[dst_s]

    h = jnp.pad(x, ((0, npad - N), (0, 0)))

    wg1 = _fold_weights(W1, gru1_wih)

    def body1(i, hh):
        q = _spmm(hh, src_s, dst_s, ew2, npad)
        wgi = lax.dynamic_index_in_dim(wg1, i, keepdims=False)
        return _gru_step(q, hh, wgi, gru1_whh, gru1_bih, gru1_bhh, br1)

    h = lax.fori_loop(0, L, body1, h)
    h = _lstm_step(h, lstm1_wih, lstm1_bih, lstm1_bhh, br1)

    wg2 = _fold_weights(W2, gru2_wih)

    def body2(i, hh):
        q = _spmm(hh, src_s, dst_s, ew2, npad)
        wgi = lax.dynamic_index_in_dim(wg2, i, keepdims=False)
        return _gru_step(q, hh, wgi, gru2_whh, gru2_bih, gru2_bhh, br2)

    h = lax.fori_loop(0, L, body2, h)
    h = _lstm_step(h, lstm2_wih, lstm2_bih, lstm2_bhh, br2)

    out = _final_linear(h, lin_w, lin_b, br2)
    return out[:N, :1]
